# baseline (speedup 1.0000x reference)
import functools

import jax
import jax.numpy as jnp
from jax import lax
from jax.experimental import pallas as pl
from jax.experimental.pallas import tpu as pltpu
from jax.experimental.pallas import tpu_sc as plsc

D_MSA = 256
D_PAIR = 192
D_STATE = 64
D_RBF = 64
TOPK = 64
NITER = 4
L = 512
NCHI = 128


def _p_body(pair_ref, bond_ref, wp_ref, wb_ref, out_ref):
    pair_blk = pair_ref[...].reshape(8192, 192)
    bond_blk = bond_ref[...]
    vals = lax.broadcasted_iota(jnp.int32, (16, 512, 8), 2)
    oh = (bond_blk[:, :, None] == vals).astype(jnp.float32)
    oh = oh.reshape(8192, 8)
    e0 = jnp.dot(pair_blk, wp_ref[...],
                 preferred_element_type=jnp.float32) + \
        jnp.dot(oh, wb_ref[...], preferred_element_type=jnp.float32)
    out_ref[...] = jnp.concatenate(
        [e0, jnp.zeros((8192, 64), jnp.float32)], axis=1)


def _precompute_e0(pair2d, bond, Wp, Wb):
    return pl.pallas_call(
        _p_body,
        grid=(32,),
        in_specs=[
            pl.BlockSpec((16, 512, 192), lambda i: (i, 0, 0)),
            pl.BlockSpec((16, 512), lambda i: (i, 0)),
            pl.BlockSpec((192, 64), lambda i: (0, 0)),
            pl.BlockSpec((8, 64), lambda i: (0, 0)),
        ],
        out_specs=pl.BlockSpec((8192, 128), lambda i: (i, 0)),
        out_shape=jax.ShapeDtypeStruct((L * L, 128), jnp.float32),
    )(pair2d, bond, Wp, Wb)


def _cross(u, v):
    u0, u1, u2 = u[:, 0:1], u[:, 1:2], u[:, 2:3]
    v0, v1, v2 = v[:, 0:1], v[:, 1:2], v[:, 2:3]
    return jnp.concatenate(
        [u1 * v2 - u2 * v1, u2 * v0 - u0 * v2, u0 * v1 - u1 * v0], axis=1)


def _a_body(p_ref, caT_ref, dist_ref, msa0_ref, state_ref, ch_ref,
            wm_ref, ws_ref,
            dnbr_ref, nbr_ref, fi_ref, h_ref, gp_ref):
    h = jnp.maximum(
        jnp.dot(msa0_ref[...], wm_ref[...],
                preferred_element_type=jnp.float32)
        + jnp.dot(state_ref[...], ws_ref[...],
                  preferred_element_type=jnp.float32), 0.0)
    h_ref[...] = jnp.concatenate(
        [h, jnp.zeros((L, D_STATE), jnp.float32)], axis=1)

    p = p_ref[...]
    ch = ch_ref[...]
    idxs = jnp.clip(ch[:, 0:4].astype(jnp.int32), 0, L - 1)
    tgt = ch[:, 4:5]
    col = lax.broadcasted_iota(jnp.int32, (NCHI, L), 1)
    Ms = [(idxs[:, c:c + 1] == col).astype(jnp.float32) for c in range(4)]
    a, b, c_, d = [jnp.dot(M, p, preferred_element_type=jnp.float32)
                   for M in Ms]
    v1 = b - a
    v2 = c_ - a
    v3 = d - a
    n = _cross(v1, v2)
    nn = jnp.sqrt(jnp.sum(n * n, axis=1, keepdims=True))
    n3 = jnp.sqrt(jnp.sum(v3 * v3, axis=1, keepdims=True))
    Q = nn * n3 + 1e-6
    S = jnp.sum(n * v3, axis=1, keepdims=True)
    chi = S / Q
    g = 2.0 * (chi - tgt)
    ct_S = g / Q
    ct_Q = -g * S / (Q * Q)
    cn = jnp.where(nn > 0.0, ct_Q * n3 / jnp.maximum(nn, 1e-30), 0.0)
    c3 = jnp.where(n3 > 0.0, ct_Q * nn / jnp.maximum(n3, 1e-30), 0.0)
    ct_n = v3 * ct_S + cn * n
    ct_v3 = n * ct_S + c3 * v3
    ct_v1 = _cross(v2, ct_n)
    ct_v2 = _cross(ct_n, v1)
    ct_a = -(ct_v1 + ct_v2 + ct_v3)
    gp = jnp.zeros((L, 3), jnp.float32)
    for M, ct in zip(Ms, (ct_a, ct_v1, ct_v2, ct_v3)):
        gp = gp + lax.dot_general(M, ct, (((0,), (0,)), ((), ())),
                                  preferred_element_type=jnp.float32)
    gp_ref[...] = gp

    acc = jnp.zeros((L, L), jnp.float32)
    for c in range(3):
        diff = p[:, c:c + 1] - caT_ref[c:c + 1, :]
        acc = acc + diff * diff
    D = jnp.sqrt(acc) + dist_ref[...]

    colf = lax.broadcasted_iota(jnp.int32, (L, L), 1).astype(jnp.float32)
    lane64 = lax.broadcasted_iota(jnp.int32, (L, TOPK), 1)

    def body(k, carry):
        W, dn_acc, id_acc = carry
        m = jnp.min(W, axis=1, keepdims=True)
        idx = jnp.min(jnp.where(W == m, colf, 1e9), axis=1, keepdims=True)
        sel = lane64 == k
        dn_acc = jnp.where(sel, m, dn_acc)
        id_acc = jnp.where(sel, idx, id_acc)
        W = jnp.where(colf == idx, jnp.inf, W)
        return W, dn_acc, id_acc

    zero64 = jnp.zeros((L, TOPK), jnp.float32)
    _, dn_acc, id_acc = lax.fori_loop(0, TOPK, body, (D, zero64, zero64))
    nbr = id_acc.astype(jnp.int32)
    dnbr_ref[...] = dn_acc
    nbr_ref[...] = nbr
    row = lax.broadcasted_iota(jnp.int32, (L, TOPK), 0)
    fi_ref[...] = nbr + L * row


def _call_a(p, caT, dist, msa0, state, ch, Wm, Ws):
    return pl.pallas_call(
        _a_body,
        out_shape=[
            jax.ShapeDtypeStruct((L, TOPK), jnp.float32),
            jax.ShapeDtypeStruct((L, TOPK), jnp.int32),
            jax.ShapeDtypeStruct((L, TOPK), jnp.int32),
            jax.ShapeDtypeStruct((L, 128), jnp.float32),
            jax.ShapeDtypeStruct((L, 3), jnp.float32),
        ],
    )(p, caT, dist, msa0, state, ch, Wm, Ws)


def _sc_gather(e0, h, fi, nbr):
    info = plsc.get_sparse_core_info()
    nw = info.num_cores * info.num_subcores
    rows = L * TOPK
    per_w = rows // nw
    chunk = 128
    nchunk = per_w // chunk
    mesh = plsc.VectorSubcoreMesh(core_axis_name="c", subcore_axis_name="s")

    @functools.partial(
        pl.kernel, mesh=mesh,
        compiler_params=pltpu.CompilerParams(use_tc_tiling_on_sc=True),
        out_type=[
            jax.ShapeDtypeStruct((rows, 128), jnp.float32),
            jax.ShapeDtypeStruct((rows, 128), jnp.float32),
        ],
        scratch_types=[
            pltpu.VMEM((chunk,), jnp.int32),
            pltpu.VMEM((chunk, 128), jnp.float32),
            pltpu.SemaphoreType.DMA,
        ],
    )
    def gk(e0_hbm, h_hbm, fi_hbm, nbr_hbm, e0g_hbm, hj_hbm,
           idx_v, rows_v, sem):
        wid = lax.axis_index("s") * info.num_cores + lax.axis_index("c")
        base = wid * per_w
        for j in range(nchunk):
            off = base + j * chunk
            pltpu.sync_copy(fi_hbm.at[pl.ds(off, chunk)], idx_v)
            pltpu.async_copy(e0_hbm.at[idx_v], rows_v, sem).wait()
            pltpu.sync_copy(rows_v, e0g_hbm.at[pl.ds(off, chunk)])
            pltpu.sync_copy(nbr_hbm.at[pl.ds(off, chunk)], idx_v)
            pltpu.async_copy(h_hbm.at[idx_v], rows_v, sem).wait()
            pltpu.sync_copy(rows_v, hj_hbm.at[pl.ds(off, chunk)])

    return gk(e0, h, fi, nbr)


_RB = 64
_GRID_B = L // _RB


def _b_body(e0g_ref, hj_ref, dnbr_ref, h_ref, xyz9_ref, gp_ref, mu_ref,
            wr_ref, wmsg_ref, wup_ref, wxyz_ref, walpha_ref,
            xyz9n_ref, hnew_ref, alpha_ref):
    nrows = _RB * TOPK
    dn3 = dnbr_ref[...][:, :, None]
    mu3 = mu_ref[...][None, :, :]
    rb = jnp.exp(-((dn3 - mu3) ** 2) / (2.0 * (20.0 / D_RBF) ** 2))
    rb = rb.reshape(nrows, D_RBF)
    e_r = jnp.dot(rb, wr_ref[...], preferred_element_type=jnp.float32)
    e = jnp.maximum(e0g_ref[:, 0:64] + e_r, 0.0)

    h_blk = h_ref[:, 0:64]
    w1 = wmsg_ref[0:64, :]
    w2 = wmsg_ref[64:128, :]
    w3 = wmsg_ref[128:192, :]
    r_row = lax.broadcasted_iota(jnp.int32, (nrows, _RB), 0)
    r_col = lax.broadcasted_iota(jnp.int32, (nrows, _RB), 1)
    R = ((r_row // TOPK) == r_col).astype(jnp.float32)
    hiw = jnp.dot(R, jnp.dot(h_blk, w1, preferred_element_type=jnp.float32),
                  preferred_element_type=jnp.float32)
    msg = jnp.maximum(
        hiw
        + jnp.dot(hj_ref[:, 0:64], w2, preferred_element_type=jnp.float32)
        + jnp.dot(e, w3, preferred_element_type=jnp.float32), 0.0)
    agg = lax.dot_general(R, msg, (((0,), (0,)), ((), ())),
                          preferred_element_type=jnp.float32) * (1.0 / TOPK)

    h_new = jnp.maximum(
        jnp.dot(h_blk, wup_ref[0:64, :], preferred_element_type=jnp.float32)
        + jnp.dot(agg, wup_ref[64:128, :],
                  preferred_element_type=jnp.float32), 0.0)
    hnew_ref[...] = h_new

    vec = jnp.dot(h_new, wxyz_ref[...], preferred_element_type=jnp.float32)
    v0 = vec[:, 0:3]
    gp = gp_ref[...]
    upd = jnp.concatenate([v0, v0 + gp, v0], axis=1)
    xyz9n_ref[...] = xyz9_ref[...] + 0.1 * upd

    alpha = jnp.dot(h_new, walpha_ref[...], preferred_element_type=jnp.float32)
    a_row = lax.broadcasted_iota(jnp.int32, (20, 10), 0)
    a_col = lax.broadcasted_iota(jnp.int32, (20, 10), 1)
    PM = ((a_row // 2) == a_col).astype(jnp.float32)
    ps = jnp.dot(alpha * alpha, PM, preferred_element_type=jnp.float32)
    inv = 1.0 / (jnp.sqrt(ps) + 1e-6)
    inv20 = lax.dot_general(inv, PM, (((1,), (1,)), ((), ())),
                            preferred_element_type=jnp.float32)
    alpha_ref[...] = alpha * inv20


def _call_b(e0g, hj, dnbr, h, xyz9, gp, mu, Wr, Wmsg, Wup, Wxyz, Walpha):
    return pl.pallas_call(
        _b_body,
        grid=(_GRID_B,),
        in_specs=[
            pl.BlockSpec((_RB * TOPK, 128), lambda i: (i, 0)),
            pl.BlockSpec((_RB * TOPK, 128), lambda i: (i, 0)),
            pl.BlockSpec((_RB, TOPK), lambda i: (i, 0)),
            pl.BlockSpec((_RB, 128), lambda i: (i, 0)),
            pl.BlockSpec((_RB, 9), lambda i: (i, 0)),
            pl.BlockSpec((_RB, 3), lambda i: (i, 0)),
            pl.BlockSpec((1, D_RBF), lambda i: (0, 0)),
            pl.BlockSpec((D_RBF, 64), lambda i: (0, 0)),
            pl.BlockSpec((192, 64), lambda i: (0, 0)),
            pl.BlockSpec((128, 64), lambda i: (0, 0)),
            pl.BlockSpec((64, 6), lambda i: (0, 0)),
            pl.BlockSpec((64, 20), lambda i: (0, 0)),
        ],
        out_specs=[
            pl.BlockSpec((_RB, 9), lambda i: (i, 0)),
            pl.BlockSpec((_RB, D_STATE), lambda i: (i, 0)),
            pl.BlockSpec((_RB, 20), lambda i: (i, 0)),
        ],
        out_shape=[
            jax.ShapeDtypeStruct((L, 9), jnp.float32),
            jax.ShapeDtypeStruct((L, D_STATE), jnp.float32),
            jax.ShapeDtypeStruct((L, 20), jnp.float32),
        ],
    )(e0g, hj, dnbr, h, xyz9, gp, mu, Wr, Wmsg, Wup, Wxyz, Walpha)


def kernel(msa, pair, xyz, state, idx, is_atom, bond_feats, dist_matrix,
           atom_frames, chirals, Wm, Ws, Wp, Wr, Wb, Wmsg, Wup, Wxyz,
           Walpha, Wquat):
    msa0 = msa[0, 0].astype(jnp.float32)
    pair3d = pair[0].astype(jnp.float32)
    bond = bond_feats[0].astype(jnp.int32)
    dist = dist_matrix[0].astype(jnp.float32)
    ch = chirals[0].astype(jnp.float32)
    st = state[0].astype(jnp.float32)
    xyz9 = xyz[0].astype(jnp.float32).reshape(L, 9)
    mu = jnp.linspace(0.0, 20.0, D_RBF).reshape(1, D_RBF)

    e0 = _precompute_e0(pair3d, bond, Wp, Wb)

    xyzs = []
    alphas = []
    for _ in range(NITER):
        p = xyz9[:, 3:6]
        caT = p.T
        dnbr, nbr, fi, h, gp = _call_a(p, caT, dist, msa0, st, ch, Wm, Ws)
        e0g, hj = _sc_gather(e0, h, fi.reshape(-1), nbr.reshape(-1))
        xyz9, st, alpha = _call_b(e0g, hj, dnbr, h, xyz9, gp, mu,
                                  Wr, Wmsg, Wup, Wxyz, Walpha)
        xyzs.append(xyz9.reshape(1, L, 3, 3))
        alphas.append(alpha.reshape(1, L, 10, 2))

    return (jnp.stack(xyzs, 0), st[None], jnp.stack(alphas, 0))


# speedup vs baseline: 1.2227x; 1.2227x over previous
import functools

import jax
import jax.numpy as jnp
from jax import lax
from jax.experimental import pallas as pl
from jax.experimental.pallas import tpu as pltpu
from jax.experimental.pallas import tpu_sc as plsc

D_MSA = 256
D_PAIR = 192
D_STATE = 64
D_RBF = 64
TOPK = 64
NITER = 4
L = 512
NCHI = 128


def _p_body(pairT_ref, bond_ref, wp_ref, wb_ref, out_ref):
    bond_blk = bond_ref[...]
    vals = lax.broadcasted_iota(jnp.int32, (16, 512, 8), 2)
    oh = (bond_blk[:, :, None] == vals).astype(jnp.float32)
    oh = oh.reshape(8192, 8)
    wb_all = jnp.dot(oh, wb_ref[...], preferred_element_type=jnp.float32)
    zpad = jnp.zeros((512, 64), jnp.float32)
    for k in range(16):
        e0k = lax.dot_general(pairT_ref[k], wp_ref[...],
                              (((0,), (0,)), ((), ())),
                              preferred_element_type=jnp.float32)
        e0k = e0k + wb_all[k * 512:(k + 1) * 512, :]
        out_ref[k * 512:(k + 1) * 512, :] = jnp.concatenate(
            [e0k, zpad], axis=1)


def _precompute_e0(pair2d, bond, Wp, Wb):
    return pl.pallas_call(
        _p_body,
        grid=(32,),
        in_specs=[
            pl.BlockSpec((16, 192, 512), lambda i: (i, 0, 0)),
            pl.BlockSpec((16, 512), lambda i: (i, 0)),
            pl.BlockSpec((192, 64), lambda i: (0, 0)),
            pl.BlockSpec((8, 64), lambda i: (0, 0)),
        ],
        out_specs=pl.BlockSpec((8192, 128), lambda i: (i, 0)),
        out_shape=jax.ShapeDtypeStruct((L * L, 128), jnp.float32),
    )(pair2d, bond, Wp, Wb)


def _cross(u, v):
    u0, u1, u2 = u[:, 0:1], u[:, 1:2], u[:, 2:3]
    v0, v1, v2 = v[:, 0:1], v[:, 1:2], v[:, 2:3]
    return jnp.concatenate(
        [u1 * v2 - u2 * v1, u2 * v0 - u0 * v2, u0 * v1 - u1 * v0], axis=1)


def _a_body(p_ref, caT_ref, dist_ref, msa0_ref, state_ref, ch_ref,
            wm_ref, ws_ref,
            dnbr_ref, nbr_ref, fi_ref, h_ref, gp_ref):
    h = jnp.maximum(
        jnp.dot(msa0_ref[...], wm_ref[...],
                preferred_element_type=jnp.float32)
        + jnp.dot(state_ref[...], ws_ref[...],
                  preferred_element_type=jnp.float32), 0.0)
    h_ref[...] = jnp.concatenate(
        [h, jnp.zeros((L, D_STATE), jnp.float32)], axis=1)

    p = p_ref[...]
    ch = ch_ref[...]
    idxs = jnp.clip(ch[:, 0:4].astype(jnp.int32), 0, L - 1)
    tgt = ch[:, 4:5]
    col = lax.broadcasted_iota(jnp.int32, (NCHI, L), 1)
    Ms = [(idxs[:, c:c + 1] == col).astype(jnp.float32) for c in range(4)]
    a, b, c_, d = [jnp.dot(M, p, preferred_element_type=jnp.float32)
                   for M in Ms]
    v1 = b - a
    v2 = c_ - a
    v3 = d - a
    n = _cross(v1, v2)
    nn = jnp.sqrt(jnp.sum(n * n, axis=1, keepdims=True))
    n3 = jnp.sqrt(jnp.sum(v3 * v3, axis=1, keepdims=True))
    Q = nn * n3 + 1e-6
    S = jnp.sum(n * v3, axis=1, keepdims=True)
    chi = S / Q
    g = 2.0 * (chi - tgt)
    ct_S = g / Q
    ct_Q = -g * S / (Q * Q)
    cn = jnp.where(nn > 0.0, ct_Q * n3 / jnp.maximum(nn, 1e-30), 0.0)
    c3 = jnp.where(n3 > 0.0, ct_Q * nn / jnp.maximum(n3, 1e-30), 0.0)
    ct_n = v3 * ct_S + cn * n
    ct_v3 = n * ct_S + c3 * v3
    ct_v1 = _cross(v2, ct_n)
    ct_v2 = _cross(ct_n, v1)
    ct_a = -(ct_v1 + ct_v2 + ct_v3)
    gp = jnp.zeros((L, 3), jnp.float32)
    for M, ct in zip(Ms, (ct_a, ct_v1, ct_v2, ct_v3)):
        gp = gp + lax.dot_general(M, ct, (((0,), (0,)), ((), ())),
                                  preferred_element_type=jnp.float32)
    gp_ref[...] = gp

    acc = jnp.zeros((L, L), jnp.float32)
    for c in range(3):
        diff = p[:, c:c + 1] - caT_ref[c:c + 1, :]
        acc = acc + diff * diff
    D = jnp.sqrt(acc) + dist_ref[...]

    colf = lax.broadcasted_iota(jnp.int32, (L, L), 1).astype(jnp.float32)
    lane64 = lax.broadcasted_iota(jnp.int32, (L, TOPK), 1)

    def body(k, carry):
        W, dn_acc, id_acc = carry
        m = jnp.min(W, axis=1, keepdims=True)
        idx = jnp.min(jnp.where(W == m, colf, 1e9), axis=1, keepdims=True)
        sel = lane64 == k
        dn_acc = jnp.where(sel, m, dn_acc)
        id_acc = jnp.where(sel, idx, id_acc)
        W = jnp.where(colf == idx, jnp.inf, W)
        return W, dn_acc, id_acc

    zero64 = jnp.zeros((L, TOPK), jnp.float32)
    _, dn_acc, id_acc = lax.fori_loop(0, TOPK, body, (D, zero64, zero64))
    nbr = id_acc.astype(jnp.int32)
    dnbr_ref[...] = dn_acc
    nbr_ref[...] = nbr
    row = lax.broadcasted_iota(jnp.int32, (L, TOPK), 0)
    fi_ref[...] = nbr + L * row


def _call_a(p, caT, dist, msa0, state, ch, Wm, Ws):
    return pl.pallas_call(
        _a_body,
        out_shape=[
            jax.ShapeDtypeStruct((L, TOPK), jnp.float32),
            jax.ShapeDtypeStruct((L, TOPK), jnp.int32),
            jax.ShapeDtypeStruct((L, TOPK), jnp.int32),
            jax.ShapeDtypeStruct((L, 128), jnp.float32),
            jax.ShapeDtypeStruct((L, 3), jnp.float32),
        ],
    )(p, caT, dist, msa0, state, ch, Wm, Ws)


def _sc_gather(e0, h, fi, nbr):
    info = plsc.get_sparse_core_info()
    nw = info.num_cores * info.num_subcores
    rows = L * TOPK
    per_w = rows // nw
    chunk = 128
    nchunk = per_w // chunk
    mesh = plsc.VectorSubcoreMesh(core_axis_name="c", subcore_axis_name="s")

    @functools.partial(
        pl.kernel, mesh=mesh,
        compiler_params=pltpu.CompilerParams(use_tc_tiling_on_sc=True),
        out_type=[
            jax.ShapeDtypeStruct((rows, 128), jnp.float32),
            jax.ShapeDtypeStruct((rows, 128), jnp.float32),
        ],
        scratch_types=[
            pltpu.VMEM((chunk,), jnp.int32),
            pltpu.VMEM((chunk, 128), jnp.float32),
            pltpu.SemaphoreType.DMA,
        ],
    )
    def gk(e0_hbm, h_hbm, fi_hbm, nbr_hbm, e0g_hbm, hj_hbm,
           idx_v, rows_v, sem):
        wid = lax.axis_index("s") * info.num_cores + lax.axis_index("c")
        base = wid * per_w
        for j in range(nchunk):
            off = base + j * chunk
            pltpu.sync_copy(fi_hbm.at[pl.ds(off, chunk)], idx_v)
            pltpu.async_copy(e0_hbm.at[idx_v], rows_v, sem).wait()
            pltpu.sync_copy(rows_v, e0g_hbm.at[pl.ds(off, chunk)])
            pltpu.sync_copy(nbr_hbm.at[pl.ds(off, chunk)], idx_v)
            pltpu.async_copy(h_hbm.at[idx_v], rows_v, sem).wait()
            pltpu.sync_copy(rows_v, hj_hbm.at[pl.ds(off, chunk)])

    return gk(e0, h, fi, nbr)


_RB = 64
_GRID_B = L // _RB


def _b_body(e0g_ref, hj_ref, dnbr_ref, h_ref, xyz9_ref, gp_ref, mu_ref,
            wr_ref, wmsg_ref, wup_ref, wxyz_ref, walpha_ref,
            xyz9n_ref, hnew_ref, alpha_ref):
    nrows = _RB * TOPK
    dn3 = dnbr_ref[...][:, :, None]
    mu3 = mu_ref[...][None, :, :]
    rb = jnp.exp(-((dn3 - mu3) ** 2) / (2.0 * (20.0 / D_RBF) ** 2))
    rb = rb.reshape(nrows, D_RBF)
    e_r = jnp.dot(rb, wr_ref[...], preferred_element_type=jnp.float32)
    e = jnp.maximum(e0g_ref[:, 0:64] + e_r, 0.0)

    h_blk = h_ref[:, 0:64]
    w1 = wmsg_ref[0:64, :]
    w2 = wmsg_ref[64:128, :]
    w3 = wmsg_ref[128:192, :]
    r_row = lax.broadcasted_iota(jnp.int32, (nrows, _RB), 0)
    r_col = lax.broadcasted_iota(jnp.int32, (nrows, _RB), 1)
    R = ((r_row // TOPK) == r_col).astype(jnp.float32)
    hiw = jnp.dot(R, jnp.dot(h_blk, w1, preferred_element_type=jnp.float32),
                  preferred_element_type=jnp.float32)
    msg = jnp.maximum(
        hiw
        + jnp.dot(hj_ref[:, 0:64], w2, preferred_element_type=jnp.float32)
        + jnp.dot(e, w3, preferred_element_type=jnp.float32), 0.0)
    agg = lax.dot_general(R, msg, (((0,), (0,)), ((), ())),
                          preferred_element_type=jnp.float32) * (1.0 / TOPK)

    h_new = jnp.maximum(
        jnp.dot(h_blk, wup_ref[0:64, :], preferred_element_type=jnp.float32)
        + jnp.dot(agg, wup_ref[64:128, :],
                  preferred_element_type=jnp.float32), 0.0)
    hnew_ref[...] = h_new

    vec = jnp.dot(h_new, wxyz_ref[...], preferred_element_type=jnp.float32)
    v0 = vec[:, 0:3]
    gp = gp_ref[...]
    upd = jnp.concatenate([v0, v0 + gp, v0], axis=1)
    xyz9n_ref[...] = xyz9_ref[...] + 0.1 * upd

    alpha = jnp.dot(h_new, walpha_ref[...], preferred_element_type=jnp.float32)
    a_row = lax.broadcasted_iota(jnp.int32, (20, 10), 0)
    a_col = lax.broadcasted_iota(jnp.int32, (20, 10), 1)
    PM = ((a_row // 2) == a_col).astype(jnp.float32)
    ps = jnp.dot(alpha * alpha, PM, preferred_element_type=jnp.float32)
    inv = 1.0 / (jnp.sqrt(ps) + 1e-6)
    inv20 = lax.dot_general(inv, PM, (((1,), (1,)), ((), ())),
                            preferred_element_type=jnp.float32)
    alpha_ref[...] = alpha * inv20


def _call_b(e0g, hj, dnbr, h, xyz9, gp, mu, Wr, Wmsg, Wup, Wxyz, Walpha):
    return pl.pallas_call(
        _b_body,
        grid=(_GRID_B,),
        in_specs=[
            pl.BlockSpec((_RB * TOPK, 128), lambda i: (i, 0)),
            pl.BlockSpec((_RB * TOPK, 128), lambda i: (i, 0)),
            pl.BlockSpec((_RB, TOPK), lambda i: (i, 0)),
            pl.BlockSpec((_RB, 128), lambda i: (i, 0)),
            pl.BlockSpec((_RB, 9), lambda i: (i, 0)),
            pl.BlockSpec((_RB, 3), lambda i: (i, 0)),
            pl.BlockSpec((1, D_RBF), lambda i: (0, 0)),
            pl.BlockSpec((D_RBF, 64), lambda i: (0, 0)),
            pl.BlockSpec((192, 64), lambda i: (0, 0)),
            pl.BlockSpec((128, 64), lambda i: (0, 0)),
            pl.BlockSpec((64, 6), lambda i: (0, 0)),
            pl.BlockSpec((64, 20), lambda i: (0, 0)),
        ],
        out_specs=[
            pl.BlockSpec((_RB, 9), lambda i: (i, 0)),
            pl.BlockSpec((_RB, D_STATE), lambda i: (i, 0)),
            pl.BlockSpec((_RB, 20), lambda i: (i, 0)),
        ],
        out_shape=[
            jax.ShapeDtypeStruct((L, 9), jnp.float32),
            jax.ShapeDtypeStruct((L, D_STATE), jnp.float32),
            jax.ShapeDtypeStruct((L, 20), jnp.float32),
        ],
    )(e0g, hj, dnbr, h, xyz9, gp, mu, Wr, Wmsg, Wup, Wxyz, Walpha)


def kernel(msa, pair, xyz, state, idx, is_atom, bond_feats, dist_matrix,
           atom_frames, chirals, Wm, Ws, Wp, Wr, Wb, Wmsg, Wup, Wxyz,
           Walpha, Wquat):
    msa0 = msa[0, 0].astype(jnp.float32)
    pairT = jnp.transpose(pair[0].astype(jnp.float32), (0, 2, 1))
    bond = bond_feats[0].astype(jnp.int32)
    dist = dist_matrix[0].astype(jnp.float32)
    ch = chirals[0].astype(jnp.float32)
    st = state[0].astype(jnp.float32)
    xyz9 = xyz[0].astype(jnp.float32).reshape(L, 9)
    mu = jnp.linspace(0.0, 20.0, D_RBF).reshape(1, D_RBF)

    e0 = _precompute_e0(pairT, bond, Wp, Wb)

    xyzs = []
    alphas = []
    for _ in range(NITER):
        p = xyz9[:, 3:6]
        caT = p.T
        dnbr, nbr, fi, h, gp = _call_a(p, caT, dist, msa0, st, ch, Wm, Ws)
        e0g, hj = _sc_gather(e0, h, fi.reshape(-1), nbr.reshape(-1))
        xyz9, st, alpha = _call_b(e0g, hj, dnbr, h, xyz9, gp, mu,
                                  Wr, Wmsg, Wup, Wxyz, Walpha)
        xyzs.append(xyz9.reshape(1, L, 3, 3))
        alphas.append(alpha.reshape(1, L, 10, 2))

    return (jnp.stack(xyzs, 0), st[None], jnp.stack(alphas, 0))


# speedup vs baseline: 1.5236x; 1.2461x over previous
import functools

import jax
import jax.numpy as jnp
from jax import lax
from jax.experimental import pallas as pl
from jax.experimental.pallas import tpu as pltpu
from jax.experimental.pallas import tpu_sc as plsc

D_MSA = 256
D_PAIR = 192
D_STATE = 64
D_RBF = 64
TOPK = 64
NITER = 4
L = 512
NCHI = 128


def _p_body(pairT_ref, bond_ref, wp_ref, wb_ref, out_ref):
    bond_blk = bond_ref[...]
    vals = lax.broadcasted_iota(jnp.int32, (16, 512, 8), 2)
    oh = (bond_blk[:, :, None] == vals).astype(jnp.float32)
    oh = oh.reshape(8192, 8)
    wb_all = jnp.dot(oh, wb_ref[...], preferred_element_type=jnp.float32)
    zpad = jnp.zeros((512, 64), jnp.float32)
    for k in range(16):
        e0k = lax.dot_general(pairT_ref[k], wp_ref[...],
                              (((0,), (0,)), ((), ())),
                              preferred_element_type=jnp.float32)
        e0k = e0k + wb_all[k * 512:(k + 1) * 512, :]
        out_ref[k * 512:(k + 1) * 512, :] = jnp.concatenate(
            [e0k, zpad], axis=1)


def _precompute_e0(pair2d, bond, Wp, Wb):
    return pl.pallas_call(
        _p_body,
        grid=(32,),
        in_specs=[
            pl.BlockSpec((16, 192, 512), lambda i: (i, 0, 0)),
            pl.BlockSpec((16, 512), lambda i: (i, 0)),
            pl.BlockSpec((192, 64), lambda i: (0, 0)),
            pl.BlockSpec((8, 64), lambda i: (0, 0)),
        ],
        out_specs=pl.BlockSpec((8192, 128), lambda i: (i, 0)),
        out_shape=jax.ShapeDtypeStruct((L * L, 128), jnp.float32),
    )(pair2d, bond, Wp, Wb)


def _cross(u, v):
    u0, u1, u2 = u[:, 0:1], u[:, 1:2], u[:, 2:3]
    v0, v1, v2 = v[:, 0:1], v[:, 1:2], v[:, 2:3]
    return jnp.concatenate(
        [u1 * v2 - u2 * v1, u2 * v0 - u0 * v2, u0 * v1 - u1 * v0], axis=1)


def _a_body(p_ref, caT_ref, dist_ref, msa0_ref, state_ref, ch_ref,
            wm_ref, ws_ref,
            dnbr_ref, nbr_ref, fi_ref, h_ref, gp_ref):
    h = jnp.maximum(
        jnp.dot(msa0_ref[...], wm_ref[...],
                preferred_element_type=jnp.float32)
        + jnp.dot(state_ref[...], ws_ref[...],
                  preferred_element_type=jnp.float32), 0.0)
    h_ref[...] = h

    p = p_ref[...]
    ch = ch_ref[...]
    idxs = jnp.clip(ch[:, 0:4].astype(jnp.int32), 0, L - 1)
    tgt = ch[:, 4:5]
    col = lax.broadcasted_iota(jnp.int32, (NCHI, L), 1)
    Ms = [(idxs[:, c:c + 1] == col).astype(jnp.float32) for c in range(4)]
    a, b, c_, d = [jnp.dot(M, p, preferred_element_type=jnp.float32)
                   for M in Ms]
    v1 = b - a
    v2 = c_ - a
    v3 = d - a
    n = _cross(v1, v2)
    nn = jnp.sqrt(jnp.sum(n * n, axis=1, keepdims=True))
    n3 = jnp.sqrt(jnp.sum(v3 * v3, axis=1, keepdims=True))
    Q = nn * n3 + 1e-6
    S = jnp.sum(n * v3, axis=1, keepdims=True)
    chi = S / Q
    g = 2.0 * (chi - tgt)
    ct_S = g / Q
    ct_Q = -g * S / (Q * Q)
    cn = jnp.where(nn > 0.0, ct_Q * n3 / jnp.maximum(nn, 1e-30), 0.0)
    c3 = jnp.where(n3 > 0.0, ct_Q * nn / jnp.maximum(n3, 1e-30), 0.0)
    ct_n = v3 * ct_S + cn * n
    ct_v3 = n * ct_S + c3 * v3
    ct_v1 = _cross(v2, ct_n)
    ct_v2 = _cross(ct_n, v1)
    ct_a = -(ct_v1 + ct_v2 + ct_v3)
    gp = jnp.zeros((L, 3), jnp.float32)
    for M, ct in zip(Ms, (ct_a, ct_v1, ct_v2, ct_v3)):
        gp = gp + lax.dot_general(M, ct, (((0,), (0,)), ((), ())),
                                  preferred_element_type=jnp.float32)
    gp_ref[...] = gp

    acc = jnp.zeros((L, L), jnp.float32)
    for c in range(3):
        diff = p[:, c:c + 1] - caT_ref[c:c + 1, :]
        acc = acc + diff * diff
    D = jnp.sqrt(acc) + dist_ref[...]

    colf = lax.broadcasted_iota(jnp.int32, (L, L), 1).astype(jnp.float32)
    lane64 = lax.broadcasted_iota(jnp.int32, (L, TOPK), 1)

    def body(k, carry):
        W, dn_acc, id_acc = carry
        m = jnp.min(W, axis=1, keepdims=True)
        idx = jnp.min(jnp.where(W == m, colf, 1e9), axis=1, keepdims=True)
        sel = lane64 == k
        dn_acc = jnp.where(sel, m, dn_acc)
        id_acc = jnp.where(sel, idx, id_acc)
        W = jnp.where(colf == idx, jnp.inf, W)
        return W, dn_acc, id_acc

    zero64 = jnp.zeros((L, TOPK), jnp.float32)
    _, dn_acc, id_acc = lax.fori_loop(0, TOPK, body, (D, zero64, zero64))
    nbr = id_acc.astype(jnp.int32)
    dnbr_ref[...] = dn_acc
    nbr_ref[...] = nbr
    row = lax.broadcasted_iota(jnp.int32, (L, TOPK), 0)
    fi_ref[...] = nbr + L * row


def _call_a(p, caT, dist, msa0, state, ch, Wm, Ws):
    return pl.pallas_call(
        _a_body,
        out_shape=[
            jax.ShapeDtypeStruct((L, TOPK), jnp.float32),
            jax.ShapeDtypeStruct((L, TOPK), jnp.int32),
            jax.ShapeDtypeStruct((L, TOPK), jnp.int32),
            jax.ShapeDtypeStruct((L, D_STATE), jnp.float32),
            jax.ShapeDtypeStruct((L, 3), jnp.float32),
        ],
    )(p, caT, dist, msa0, state, ch, Wm, Ws)


def _sc_gather_e0(e0, fi):
    info = plsc.get_sparse_core_info()
    nw = info.num_cores * info.num_subcores
    rows = L * TOPK
    per_w = rows // nw
    mesh = plsc.VectorSubcoreMesh(core_axis_name="c", subcore_axis_name="s")

    @functools.partial(
        pl.kernel, mesh=mesh,
        compiler_params=pltpu.CompilerParams(use_tc_tiling_on_sc=True),
        out_type=jax.ShapeDtypeStruct((rows, 128), jnp.float32),
        scratch_types=[
            pltpu.VMEM((per_w,), jnp.int32),
            pltpu.VMEM((256, 128), jnp.float32),
            pltpu.VMEM((256, 128), jnp.float32),
            pltpu.SemaphoreType.DMA,
            pltpu.SemaphoreType.DMA,
            pltpu.SemaphoreType.DMA,
        ],
    )
    def gk(e0_hbm, fi_hbm, out_hbm, idx_v, buf_a, buf_b,
           sem_g, sem_wa, sem_wb):
        wid = lax.axis_index("s") * info.num_cores + lax.axis_index("c")
        base = wid * per_w
        pltpu.sync_copy(fi_hbm.at[pl.ds(base, per_w)], idx_v)
        bufs = (buf_a, buf_b)
        wsems = (sem_wa, sem_wb)
        for r in range(4):
            buf = bufs[r % 2]
            if r >= 2:
                pltpu.make_async_copy(
                    buf, out_hbm.at[pl.ds(base + (r - 2) * 256, 256)],
                    wsems[r % 2]).wait()
            g1 = pltpu.make_async_copy(
                e0_hbm.at[idx_v.at[pl.ds(r * 256, 128)]],
                buf.at[pl.ds(0, 128)], sem_g)
            g2 = pltpu.make_async_copy(
                e0_hbm.at[idx_v.at[pl.ds(r * 256 + 128, 128)]],
                buf.at[pl.ds(128, 128)], sem_g)
            g1.start()
            g2.start()
            g1.wait()
            g2.wait()
            pltpu.make_async_copy(
                buf, out_hbm.at[pl.ds(base + r * 256, 256)],
                wsems[r % 2]).start()
        for r in (2, 3):
            pltpu.make_async_copy(
                bufs[r % 2], out_hbm.at[pl.ds(base + r * 256, 256)],
                wsems[r % 2]).wait()

    return gk(e0, fi)


_RB = 64
_GRID_B = L // _RB


def _b_body(e0g_ref, dnbr_ref, nbr_ref, h_ref, xyz9_ref, gp_ref, mu_ref,
            wr_ref, wmsg_ref, wup_ref, wxyz_ref, walpha_ref,
            xyz9n_ref, hnew_ref, alpha_ref):
    nrows = _RB * TOPK
    dn3 = dnbr_ref[...][:, :, None]
    mu3 = mu_ref[...][None, :, :]
    rb = jnp.exp(-((dn3 - mu3) ** 2) / (2.0 * (20.0 / D_RBF) ** 2))
    rb = rb.reshape(nrows, D_RBF)
    e_r = jnp.dot(rb, wr_ref[...], preferred_element_type=jnp.float32)
    e = jnp.maximum(e0g_ref[:, 0:64] + e_r, 0.0)

    pid = pl.program_id(0)
    h_all = h_ref[...]
    h_blk = h_ref[pl.ds(pid * _RB, _RB), :]
    w1 = wmsg_ref[0:64, :]
    w2 = wmsg_ref[64:128, :]
    w3 = wmsg_ref[128:192, :]
    nbr_blk = nbr_ref[...]
    jota = lax.broadcasted_iota(jnp.int32, (_RB, TOPK, L), 2)
    OH = (nbr_blk[:, :, None] == jota).astype(jnp.float32).reshape(nrows, L)
    hw2 = jnp.dot(h_all, w2, preferred_element_type=jnp.float32)
    hjw = jnp.dot(OH, hw2, preferred_element_type=jnp.float32)
    r_row = lax.broadcasted_iota(jnp.int32, (nrows, _RB), 0)
    r_col = lax.broadcasted_iota(jnp.int32, (nrows, _RB), 1)
    R = ((r_row // TOPK) == r_col).astype(jnp.float32)
    hiw = jnp.dot(R, jnp.dot(h_blk, w1, preferred_element_type=jnp.float32),
                  preferred_element_type=jnp.float32)
    msg = jnp.maximum(
        hiw + hjw
        + jnp.dot(e, w3, preferred_element_type=jnp.float32), 0.0)
    agg = lax.dot_general(R, msg, (((0,), (0,)), ((), ())),
                          preferred_element_type=jnp.float32) * (1.0 / TOPK)

    h_new = jnp.maximum(
        jnp.dot(h_blk, wup_ref[0:64, :], preferred_element_type=jnp.float32)
        + jnp.dot(agg, wup_ref[64:128, :],
                  preferred_element_type=jnp.float32), 0.0)
    hnew_ref[...] = h_new

    vec = jnp.dot(h_new, wxyz_ref[...], preferred_element_type=jnp.float32)
    v0 = vec[:, 0:3]
    gp = gp_ref[...]
    upd = jnp.concatenate([v0, v0 + gp, v0], axis=1)
    xyz9n_ref[...] = xyz9_ref[...] + 0.1 * upd

    alpha = jnp.dot(h_new, walpha_ref[...], preferred_element_type=jnp.float32)
    a_row = lax.broadcasted_iota(jnp.int32, (20, 10), 0)
    a_col = lax.broadcasted_iota(jnp.int32, (20, 10), 1)
    PM = ((a_row // 2) == a_col).astype(jnp.float32)
    ps = jnp.dot(alpha * alpha, PM, preferred_element_type=jnp.float32)
    inv = 1.0 / (jnp.sqrt(ps) + 1e-6)
    inv20 = lax.dot_general(inv, PM, (((1,), (1,)), ((), ())),
                            preferred_element_type=jnp.float32)
    alpha_ref[...] = alpha * inv20


def _call_b(e0g, dnbr, nbr, h, xyz9, gp, mu, Wr, Wmsg, Wup, Wxyz, Walpha):
    return pl.pallas_call(
        _b_body,
        grid=(_GRID_B,),
        in_specs=[
            pl.BlockSpec((_RB * TOPK, 128), lambda i: (i, 0)),
            pl.BlockSpec((_RB, TOPK), lambda i: (i, 0)),
            pl.BlockSpec((_RB, TOPK), lambda i: (i, 0)),
            pl.BlockSpec((L, D_STATE), lambda i: (0, 0)),
            pl.BlockSpec((_RB, 9), lambda i: (i, 0)),
            pl.BlockSpec((_RB, 3), lambda i: (i, 0)),
            pl.BlockSpec((1, D_RBF), lambda i: (0, 0)),
            pl.BlockSpec((D_RBF, 64), lambda i: (0, 0)),
            pl.BlockSpec((192, 64), lambda i: (0, 0)),
            pl.BlockSpec((128, 64), lambda i: (0, 0)),
            pl.BlockSpec((64, 6), lambda i: (0, 0)),
            pl.BlockSpec((64, 20), lambda i: (0, 0)),
        ],
        out_specs=[
            pl.BlockSpec((_RB, 9), lambda i: (i, 0)),
            pl.BlockSpec((_RB, D_STATE), lambda i: (i, 0)),
            pl.BlockSpec((_RB, 20), lambda i: (i, 0)),
        ],
        out_shape=[
            jax.ShapeDtypeStruct((L, 9), jnp.float32),
            jax.ShapeDtypeStruct((L, D_STATE), jnp.float32),
            jax.ShapeDtypeStruct((L, 20), jnp.float32),
        ],
    )(e0g, dnbr, nbr, h, xyz9, gp, mu, Wr, Wmsg, Wup, Wxyz, Walpha)


def kernel(msa, pair, xyz, state, idx, is_atom, bond_feats, dist_matrix,
           atom_frames, chirals, Wm, Ws, Wp, Wr, Wb, Wmsg, Wup, Wxyz,
           Walpha, Wquat):
    msa0 = msa[0, 0].astype(jnp.float32)
    pairT = jnp.transpose(pair[0].astype(jnp.float32), (0, 2, 1))
    bond = bond_feats[0].astype(jnp.int32)
    dist = dist_matrix[0].astype(jnp.float32)
    ch = chirals[0].astype(jnp.float32)
    st = state[0].astype(jnp.float32)
    xyz9 = xyz[0].astype(jnp.float32).reshape(L, 9)
    mu = jnp.linspace(0.0, 20.0, D_RBF).reshape(1, D_RBF)

    e0 = _precompute_e0(pairT, bond, Wp, Wb)

    xyzs = []
    alphas = []
    for _ in range(NITER):
        p = xyz9[:, 3:6]
        caT = p.T
        dnbr, nbr, fi, h, gp = _call_a(p, caT, dist, msa0, st, ch, Wm, Ws)
        e0g = _sc_gather_e0(e0, fi.reshape(-1))
        xyz9, st, alpha = _call_b(e0g, dnbr, nbr, h, xyz9, gp, mu,
                                  Wr, Wmsg, Wup, Wxyz, Walpha)
        xyzs.append(xyz9.reshape(1, L, 3, 3))
        alphas.append(alpha.reshape(1, L, 10, 2))

    return (jnp.stack(xyzs, 0), st[None], jnp.stack(alphas, 0))
